# baseline (speedup 1.0000x reference)
import functools

import jax
import jax.numpy as jnp
from jax import lax
from jax.experimental import pallas as pl
from jax.experimental.pallas import tpu as pltpu
from jax.experimental.pallas import tpu_sc as plsc

NUM_CODES = 8192
DIM = 256
COMMIT = 0.25
BT = 128
CH = 128
NCH = NUM_CODES // CH
NUM_TOKENS = 8192


def _vq_body(x_ref, e_ref, xsq_ref, esq_ref, idx_ref, dsum_ref, dist_ref):
    t = pl.program_id(0)
    mm2 = lax.dot_general(
        x_ref[...] * -2.0, e_ref[...], (((1,), (1,)), ((), ())),
        preferred_element_type=jnp.float32)
    dist_ref[...] = (xsq_ref[...] + esq_ref[...]) + mm2

    def chunk(c, carry):
        m, mi = carry
        d = dist_ref[:, pl.ds(pl.multiple_of(c * CH, CH), CH)]
        upd = d < m
        return jnp.where(upd, d, m), jnp.where(upd, c, mi)

    m0 = dist_ref[:, :CH]
    mi0 = jnp.zeros((BT, CH), jnp.int32)
    m, mi = lax.fori_loop(1, NCH, chunk, (m0, mi0))
    rowmin = jnp.min(m, axis=1, keepdims=True)
    lane = lax.broadcasted_iota(jnp.int32, (BT, CH), 1)
    gidx = mi * CH + lane
    idx = jnp.min(jnp.where(m == rowmin, gidx, NUM_CODES),
                  axis=1, keepdims=True)
    idx_ref[...] = idx
    partial = jnp.sum(rowmin)

    @pl.when(t == 0)
    def _():
        dsum_ref[0, 0] = partial

    @pl.when(t != 0)
    def _():
        dsum_ref[0, 0] += partial


_vq_call = pl.pallas_call(
    _vq_body,
    grid=(NUM_TOKENS // BT,),
    in_specs=[
        pl.BlockSpec((BT, DIM), lambda t: (t, 0)),
        pl.BlockSpec((NUM_CODES, DIM), lambda t: (0, 0)),
        pl.BlockSpec((BT, 1), lambda t: (t, 0)),
        pl.BlockSpec((1, NUM_CODES), lambda t: (0, 0)),
    ],
    out_specs=[
        pl.BlockSpec((BT, 1), lambda t: (t, 0)),
        pl.BlockSpec(memory_space=pltpu.SMEM, block_shape=(1, 1),
                     index_map=lambda t: (0, 0)),
    ],
    out_shape=[
        jax.ShapeDtypeStruct((NUM_TOKENS, 1), jnp.int32),
        jax.ShapeDtypeStruct((1, 1), jnp.float32),
    ],
    scratch_shapes=[pltpu.VMEM((BT, NUM_CODES), jnp.float32)],
)


_NW = 32
_BPW = NUM_TOKENS // _NW

@functools.lru_cache(maxsize=1)
def _sc_gather_fn():
    mesh = plsc.VectorSubcoreMesh(
        core_axis_name="c", subcore_axis_name="s",
        num_cores=2, num_subcores=16)

    @functools.partial(
        pl.kernel,
        out_type=jax.ShapeDtypeStruct((NUM_TOKENS, DIM), jnp.float32),
        mesh=mesh,
        scratch_types=[
            pltpu.VMEM((_BPW,), jnp.int32),
            pltpu.VMEM((_BPW, DIM), jnp.float32),
            pltpu.SemaphoreType.DMA,
        ],
    )
    def _sc_gather(table_hbm, idx_hbm, out_hbm, idx_v, rows_v, sem):
        wid = lax.axis_index("s") * 2 + lax.axis_index("c")
        base = wid * _BPW
        pltpu.sync_copy(idx_hbm.at[pl.ds(base, _BPW)], idx_v)
        pltpu.async_copy(table_hbm.at[idx_v], rows_v, sem).wait()
        pltpu.sync_copy(rows_v, out_hbm.at[pl.ds(base, _BPW)])

    return _sc_gather


def kernel(inputs, emb_weight):
    B, C, H, W = inputs.shape
    flat = jnp.transpose(inputs, (0, 2, 3, 1)).reshape(-1, DIM)
    xsq = jnp.sum(flat ** 2, axis=1, keepdims=True)
    esq = jnp.sum(emb_weight ** 2, axis=1)
    idx2, dsum = _vq_call(flat, emb_weight, xsq, esq.reshape(1, NUM_CODES))
    rows = _sc_gather_fn()(emb_weight, idx2.reshape(NUM_TOKENS))
    quantized = jnp.transpose(rows.reshape(B, H, W, C), (0, 3, 1, 2))
    loss = dsum[0, 0] * ((1.0 + COMMIT) / inputs.size)
    return (quantized, loss, idx2)


# speedup vs baseline: 1.3343x; 1.3343x over previous
import functools

import jax
import jax.numpy as jnp
from jax import lax
from jax.experimental import pallas as pl
from jax.experimental.pallas import tpu as pltpu
from jax.experimental.pallas import tpu_sc as plsc

NUM_CODES = 8192
DIM = 256
COMMIT = 0.25
BT = 128
CH = 128
NCH = NUM_CODES // CH
NUM_TOKENS = 8192


def _vq_body(x_ref, e_ref, xsq_ref, esq_ref, idx_ref, dsum_ref):
    t = pl.program_id(0)
    mm2 = lax.dot_general(
        x_ref[...] * -2.0, e_ref[...], (((1,), (1,)), ((), ())),
        preferred_element_type=jnp.float32)
    dist = (xsq_ref[...] + esq_ref[...]) + mm2

    m = lax.slice(dist, (0, 0), (BT, CH))
    mi = jnp.zeros((BT, CH), jnp.int32)
    for c in range(1, NCH):
        d = lax.slice(dist, (0, c * CH), (BT, (c + 1) * CH))
        upd = d < m
        m = jnp.where(upd, d, m)
        mi = jnp.where(upd, jnp.int32(c), mi)
    rowmin = jnp.min(m, axis=1, keepdims=True)
    lane = lax.broadcasted_iota(jnp.int32, (BT, CH), 1)
    gidx = mi * CH + lane
    idx = jnp.min(jnp.where(m == rowmin, gidx, NUM_CODES),
                  axis=1, keepdims=True)
    idx_ref[...] = idx
    partial = jnp.sum(rowmin)

    @pl.when(t == 0)
    def _():
        dsum_ref[0, 0] = partial

    @pl.when(t != 0)
    def _():
        dsum_ref[0, 0] += partial


_vq_call = pl.pallas_call(
    _vq_body,
    grid=(NUM_TOKENS // BT,),
    in_specs=[
        pl.BlockSpec((BT, DIM), lambda t: (t, 0)),
        pl.BlockSpec((NUM_CODES, DIM), lambda t: (0, 0)),
        pl.BlockSpec((BT, 1), lambda t: (t, 0)),
        pl.BlockSpec((1, NUM_CODES), lambda t: (0, 0)),
    ],
    out_specs=[
        pl.BlockSpec((BT, 1), lambda t: (t, 0)),
        pl.BlockSpec(memory_space=pltpu.SMEM, block_shape=(1, 1),
                     index_map=lambda t: (0, 0)),
    ],
    out_shape=[
        jax.ShapeDtypeStruct((NUM_TOKENS, 1), jnp.int32),
        jax.ShapeDtypeStruct((1, 1), jnp.float32),
    ],
)


_NW = 32
_BPW = NUM_TOKENS // _NW

@functools.lru_cache(maxsize=1)
def _sc_gather_fn():
    mesh = plsc.VectorSubcoreMesh(
        core_axis_name="c", subcore_axis_name="s",
        num_cores=2, num_subcores=16)

    @functools.partial(
        pl.kernel,
        out_type=jax.ShapeDtypeStruct((NUM_TOKENS, DIM), jnp.float32),
        mesh=mesh,
        scratch_types=[
            pltpu.VMEM((_BPW,), jnp.int32),
            pltpu.VMEM((_BPW, DIM), jnp.float32),
            pltpu.SemaphoreType.DMA,
        ],
    )
    def _sc_gather(table_hbm, idx_hbm, out_hbm, idx_v, rows_v, sem):
        wid = lax.axis_index("s") * 2 + lax.axis_index("c")
        base = wid * _BPW
        pltpu.sync_copy(idx_hbm.at[pl.ds(base, _BPW)], idx_v)
        pltpu.async_copy(table_hbm.at[idx_v], rows_v, sem).wait()
        pltpu.sync_copy(rows_v, out_hbm.at[pl.ds(base, _BPW)])

    return _sc_gather


def kernel(inputs, emb_weight):
    B, C, H, W = inputs.shape
    flat = jnp.transpose(inputs, (0, 2, 3, 1)).reshape(-1, DIM)
    xsq = jnp.sum(flat ** 2, axis=1, keepdims=True)
    esq = jnp.sum(emb_weight ** 2, axis=1)
    idx2, dsum = _vq_call(flat, emb_weight, xsq, esq.reshape(1, NUM_CODES))
    rows = _sc_gather_fn()(emb_weight, idx2.reshape(NUM_TOKENS))
    quantized = jnp.transpose(rows.reshape(B, H, W, C), (0, 3, 1, 2))
    loss = dsum[0, 0] * ((1.0 + COMMIT) / inputs.size)
    return (quantized, loss, idx2)


# speedup vs baseline: 1.6413x; 1.2301x over previous
import functools

import jax
import jax.numpy as jnp
from jax import lax
from jax.experimental import pallas as pl
from jax.experimental.pallas import tpu as pltpu
from jax.experimental.pallas import tpu_sc as plsc

NUM_CODES = 8192
DIM = 256
COMMIT = 0.25
BT = 256
HT = 128
CH = 128
NCH = NUM_CODES // CH
NUM_TOKENS = 8192


def _half_argmin(dist_h):
    m = lax.slice(dist_h, (0, 0), (HT, CH))
    for c in range(1, NCH):
        m = jnp.minimum(m, lax.slice(dist_h, (0, c * CH), (HT, (c + 1) * CH)))
    rowmin = jnp.min(m, axis=1, keepdims=True)
    rb = jnp.broadcast_to(rowmin, (HT, CH))
    mi = jnp.full((HT, CH), float(NCH), jnp.float32)
    for c in range(NCH):
        d = lax.slice(dist_h, (0, c * CH), (HT, (c + 1) * CH))
        mi = jnp.minimum(mi, jnp.where(d == rb, float(c), float(NCH)))
    lane = lax.broadcasted_iota(jnp.int32, (HT, CH), 1)
    gidx = mi.astype(jnp.int32) * CH + lane
    idx = jnp.min(jnp.where(m == rowmin, gidx, NUM_CODES),
                  axis=1, keepdims=True)
    return idx, rowmin


def _vq_body(x_ref, e_ref, xsq_ref, esq_ref, idx_ref, dsum_ref):
    t = pl.program_id(0)
    mm2 = lax.dot_general(
        x_ref[...] * -2.0, e_ref[...], (((1,), (1,)), ((), ())),
        preferred_element_type=jnp.float32)
    dist = (xsq_ref[...] + esq_ref[...]) + mm2

    idx0, rm0 = _half_argmin(lax.slice(dist, (0, 0), (HT, NUM_CODES)))
    idx1, rm1 = _half_argmin(lax.slice(dist, (HT, 0), (BT, NUM_CODES)))
    idx_ref[...] = lax.concatenate([idx0, idx1], 0)
    partial = jnp.sum(rm0) + jnp.sum(rm1)

    @pl.when(t == 0)
    def _():
        dsum_ref[0, 0] = partial

    @pl.when(t != 0)
    def _():
        dsum_ref[0, 0] += partial


_vq_call = pl.pallas_call(
    _vq_body,
    grid=(NUM_TOKENS // BT,),
    in_specs=[
        pl.BlockSpec((BT, DIM), lambda t: (t, 0)),
        pl.BlockSpec((NUM_CODES, DIM), lambda t: (0, 0)),
        pl.BlockSpec((BT, 1), lambda t: (t, 0)),
        pl.BlockSpec((1, NUM_CODES), lambda t: (0, 0)),
    ],
    out_specs=[
        pl.BlockSpec((BT, 1), lambda t: (t, 0)),
        pl.BlockSpec(memory_space=pltpu.SMEM, block_shape=(1, 1),
                     index_map=lambda t: (0, 0)),
    ],
    out_shape=[
        jax.ShapeDtypeStruct((NUM_TOKENS, 1), jnp.int32),
        jax.ShapeDtypeStruct((1, 1), jnp.float32),
    ],
)


_NW = 32
_BPW = NUM_TOKENS // _NW

@functools.lru_cache(maxsize=1)
def _sc_gather_fn():
    mesh = plsc.VectorSubcoreMesh(
        core_axis_name="c", subcore_axis_name="s",
        num_cores=2, num_subcores=16)

    @functools.partial(
        pl.kernel,
        out_type=jax.ShapeDtypeStruct((NUM_TOKENS, DIM), jnp.float32),
        mesh=mesh,
        scratch_types=[
            pltpu.VMEM((_BPW,), jnp.int32),
            pltpu.VMEM((_BPW, DIM), jnp.float32),
            pltpu.SemaphoreType.DMA,
        ],
    )
    def _sc_gather(table_hbm, idx_hbm, out_hbm, idx_v, rows_v, sem):
        wid = lax.axis_index("s") * 2 + lax.axis_index("c")
        base = wid * _BPW
        pltpu.sync_copy(idx_hbm.at[pl.ds(base, _BPW)], idx_v)
        pltpu.async_copy(table_hbm.at[idx_v], rows_v, sem).wait()
        pltpu.sync_copy(rows_v, out_hbm.at[pl.ds(base, _BPW)])

    return _sc_gather


def kernel(inputs, emb_weight):
    B, C, H, W = inputs.shape
    flat = jnp.transpose(inputs, (0, 2, 3, 1)).reshape(-1, DIM)
    xsq = jnp.sum(flat ** 2, axis=1, keepdims=True)
    esq = jnp.sum(emb_weight ** 2, axis=1)
    idx2, dsum = _vq_call(flat, emb_weight, xsq, esq.reshape(1, NUM_CODES))
    rows = _sc_gather_fn()(emb_weight, idx2.reshape(NUM_TOKENS))
    quantized = jnp.transpose(rows.reshape(B, H, W, C), (0, 3, 1, 2))
    loss = dsum[0, 0] * ((1.0 + COMMIT) / inputs.size)
    return (quantized, loss, idx2)
